# Initial kernel scaffold; baseline (speedup 1.0000x reference)
import jax
import jax.numpy as jnp
from jax.experimental import pallas as pl


def kernel(x, weights):
    raise NotImplementedError("write your pallas kernel here")


# speedup vs baseline: 2.1321x; 2.1321x over previous
import jax
import jax.numpy as jnp
from jax.experimental import pallas as pl

_N = 8192
_K = 4096
_R = 64
_C = 128
_B = 128


def _mask_body(w_ref, mask_ref):
    v = jnp.abs(w_ref[...])
    u = jax.lax.bitcast_convert_type(v, jnp.int32)

    def step(i, t):
        cand = t | jax.lax.shift_left(jnp.int32(1), jnp.int32(30) - i)
        cnt = jnp.sum(jnp.where(u >= cand, jnp.int32(1), jnp.int32(0)))
        return jax.lax.select(cnt >= _K, cand, t)

    t = jax.lax.fori_loop(0, 31, step, jnp.int32(0), unroll=True)

    gt = u > t
    eq = u == t
    n_gt = jnp.sum(jnp.where(gt, jnp.int32(1), jnp.int32(0)))
    ties = (_K - n_gt).astype(jnp.float32)

    eqf = jnp.where(eq, jnp.float32(1.0), jnp.float32(0.0))
    jj = jax.lax.broadcasted_iota(jnp.int32, (_C, _C), 0)
    cc = jax.lax.broadcasted_iota(jnp.int32, (_C, _C), 1)
    tri_c = jnp.where(jj < cc, jnp.float32(1.0), jnp.float32(0.0))
    inrow = jnp.dot(eqf, tri_c, preferred_element_type=jnp.float32)
    rowsum = jnp.sum(eqf, axis=1, keepdims=True)
    r0 = jax.lax.broadcasted_iota(jnp.int32, (_R, _R), 0)
    r1 = jax.lax.broadcasted_iota(jnp.int32, (_R, _R), 1)
    tri_r = jnp.where(r1 < r0, jnp.float32(1.0), jnp.float32(0.0))
    rowpre = jnp.dot(tri_r, rowsum, preferred_element_type=jnp.float32)
    prefix = inrow + rowpre

    keep = gt | (eq & (prefix < ties))
    mask_ref[...] = jnp.where(keep, jnp.float32(1.0), jnp.float32(0.0))


def _mul_body(x_ref, m_ref, o_ref):
    o_ref[...] = x_ref[...] * m_ref[...]


def kernel(x, weights):
    w2 = weights.reshape(_R, _C)
    mask2 = pl.pallas_call(
        _mask_body,
        out_shape=jax.ShapeDtypeStruct((_R, _C), jnp.float32),
    )(w2)
    mask = mask2.reshape(_N)

    rows_per_blk = 16
    sel = pl.pallas_call(
        _mul_body,
        grid=(_B // rows_per_blk,),
        in_specs=[
            pl.BlockSpec((rows_per_blk, _N), lambda i: (i, 0)),
            pl.BlockSpec((1, _N), lambda i: (0, 0)),
        ],
        out_specs=pl.BlockSpec((rows_per_blk, _N), lambda i: (i, 0)),
        out_shape=jax.ShapeDtypeStruct((_B, _N), jnp.float32),
    )(x, mask.reshape(1, _N))
    return (sel, mask)
